# Initial kernel scaffold; baseline (speedup 1.0000x reference)
import jax
import jax.numpy as jnp
from jax.experimental import pallas as pl


def kernel(x, edge_index, edge_weight, W_init, b_init, W_gcn, W_final, b_final):
    raise NotImplementedError("write your pallas kernel here")


# speedup vs baseline: 5.7079x; 5.7079x over previous
import functools
import math

import jax
import jax.numpy as jnp
from jax import lax
from jax.experimental import pallas as pl
from jax.experimental.pallas import tpu as pltpu
from jax.experimental.pallas import tpu_sc as plsc

N = 10000
E = 320000
D = 128
L = 4
ALPHA = 0.1
THETA = 0.5

NV = 10112
NT = 32
FPT = D // NT
CB = 2000
NCB = E // CB
EPW = E // NT
DCH = EPW // CB

_mesh = plsc.VectorSubcoreMesh(core_axis_name="c", subcore_axis_name="s")
_f32 = jnp.float32
_sc_params = pltpu.CompilerParams(needs_layout_passes=False)


@functools.partial(
    pl.kernel,
    mesh=_mesh,
    out_type=jax.ShapeDtypeStruct((NT, NV), _f32),
    scratch_types=[
        pltpu.VMEM((NV,), _f32),
        pltpu.VMEM((CB,), jnp.int32)
    ],
    compiler_params=_sc_params,
)
def _deg_kernel(dst_hbm, out_hbm, acc, didx):
    c = lax.axis_index("c")
    s = lax.axis_index("s")
    wid = c * 16 + s
    base = wid * EPW

    def _zero(i, carry):
        acc[pl.ds(i * 16, 16)] = jnp.zeros((16,), _f32)
        return carry

    lax.fori_loop(0, NV // 16, _zero, 0)
    ones16 = jnp.full((16,), 1.0, _f32)

    def _chunk(k, carry):
        pltpu.sync_copy(dst_hbm.at[pl.ds(base + k * CB, CB)], didx)

        def _edge(t, carry2):
            d16 = didx[pl.ds(t * 16, 16)]
            plsc.addupdate_scatter(acc, [d16], ones16)
            return carry2

        lax.fori_loop(0, CB // 16, _edge, 0)
        return carry

    lax.fori_loop(0, DCH, _chunk, 0)
    pltpu.sync_copy(acc, out_hbm.at[wid])


@functools.partial(
    pl.kernel,
    mesh=_mesh,
    out_type=jax.ShapeDtypeStruct((D, NV), _f32),
    scratch_types=[
        pltpu.VMEM((FPT, NV), _f32),
        pltpu.VMEM((FPT, NV), _f32),
        pltpu.VMEM((CB,), jnp.int32),
        pltpu.VMEM((CB,), jnp.int32),
        pltpu.VMEM((CB,), jnp.int32),
        pltpu.VMEM((CB,), jnp.int32),
        pltpu.SemaphoreType.DMA,
        pltpu.SemaphoreType.DMA,
        pltpu.SemaphoreType.DMA,
        pltpu.SemaphoreType.DMA,
    ],
    compiler_params=_sc_params,
)
def _prop_kernel(g_hbm, src_hbm, dst_hbm, out_hbm,
                 gv, acc, s0, s1, d0, d1, ss0, ss1, ds0, ds1):
    c = lax.axis_index("c")
    s = lax.axis_index("s")
    wid = c * 16 + s
    sbufs = (s0, s1)
    dbufs = (d0, d1)
    ssems = (ss0, ss1)
    dsems = (ds0, ds1)

    pltpu.sync_copy(g_hbm.at[pl.ds(wid * FPT, FPT)], gv)

    def _zero(i, carry):
        for f in range(FPT):
            acc[f, pl.ds(i * 16, 16)] = jnp.zeros((16,), _f32)
        return carry

    lax.fori_loop(0, NV // 16, _zero, 0)

    f16s = [jnp.full((16,), f, jnp.int32) for f in range(FPT)]

    for b in range(2):
        pltpu.make_async_copy(
            src_hbm.at[pl.ds(b * CB, CB)], sbufs[b], ssems[b]).start()
        pltpu.make_async_copy(
            dst_hbm.at[pl.ds(b * CB, CB)], dbufs[b], dsems[b]).start()

    def _consume(b, k):
        pltpu.make_async_copy(
            src_hbm.at[pl.ds(k * CB, CB)], sbufs[b], ssems[b]).wait()
        pltpu.make_async_copy(
            dst_hbm.at[pl.ds(k * CB, CB)], dbufs[b], dsems[b]).wait()

        def _edge(t, carry2):
            s16 = sbufs[b][pl.ds(t * 16, 16)]
            d16 = dbufs[b][pl.ds(t * 16, 16)]
            for f in range(FPT):
                val = plsc.load_gather(gv, [f16s[f], s16])
                plsc.addupdate_scatter(acc, [f16s[f], d16], val)
            return carry2

        lax.fori_loop(0, CB // 16, _edge, 0)

    def _pair(j, carry):
        base = j * 2
        for b in range(2):
            k = base + b
            _consume(b, k)
            pltpu.make_async_copy(
                src_hbm.at[pl.ds((k + 2) * CB, CB)], sbufs[b],
                ssems[b]).start()
            pltpu.make_async_copy(
                dst_hbm.at[pl.ds((k + 2) * CB, CB)], dbufs[b],
                dsems[b]).start()
        return carry

    lax.fori_loop(0, NCB // 2 - 1, _pair, 0)
    for b in range(2):
        _consume(b, NCB - 2 + b)

    pltpu.sync_copy(acc, out_hbm.at[pl.ds(wid * FPT, FPT)])


_GRID = NV // 128


def _colT_spec():
    return pl.BlockSpec((D, 128), lambda i: (0, i))


def _row_spec():
    return pl.BlockSpec((128, D), lambda i: (i, 0))


def _dinv_spec():
    return pl.BlockSpec((1, 128), lambda i: (0, i))


def _full_spec(shape):
    nd = len(shape)
    return pl.BlockSpec(shape, lambda i: (0,) * nd)


def _init_body(x_ref, w_ref, b_ref, dg_ref, x0t_ref, gt_ref, dv_ref):
    x0 = jnp.dot(x_ref[...], w_ref[...],
                 preferred_element_type=_f32) + b_ref[...]
    x0t = x0.T
    deg = jnp.sum(dg_ref[...], axis=0) + 1.0
    dinv = (1.0 / jnp.sqrt(deg))[None, :]
    x0t_ref[...] = x0t
    dv_ref[...] = dinv
    gt_ref[...] = x0t * dinv


def _init_stage(x, w, b, degp):
    return pl.pallas_call(
        _init_body,
        grid=(_GRID,),
        in_specs=[_row_spec(), _full_spec((D, D)), _full_spec((1, D)),
                  pl.BlockSpec((NT, 128), lambda i: (0, i))],
        out_specs=[_colT_spec(), _colT_spec(), _dinv_spec()],
        out_shape=[jax.ShapeDtypeStruct((D, NV), _f32),
                   jax.ShapeDtypeStruct((D, NV), _f32),
                   jax.ShapeDtypeStruct((1, NV), _f32)],
    )(x, w, b, degp)


def _mid_layer(acc_ref, gt_ref, x0t_ref, dv_ref, w_ref, beta):
    dinv = dv_ref[...]
    p = (acc_ref[...] + gt_ref[...]) * dinv
    z = (1.0 - ALPHA) * p + ALPHA * x0t_ref[...]
    wz = lax.dot_general(w_ref[...], z, (((0,), (0,)), ((), ())),
                         preferred_element_type=_f32)
    return jnp.maximum((1.0 - beta) * z + beta * wz, 0.0)


def _layer_body(acc_ref, gt_ref, x0t_ref, dv_ref, w_ref, gout_ref, *, beta):
    h = _mid_layer(acc_ref, gt_ref, x0t_ref, dv_ref, w_ref, beta)
    gout_ref[...] = h * dv_ref[...]


def _layer_stage(acct, gt, x0t, dv, w, beta):
    return pl.pallas_call(
        functools.partial(_layer_body, beta=beta),
        grid=(_GRID,),
        in_specs=[_colT_spec(), _colT_spec(), _colT_spec(), _dinv_spec(),
                  _full_spec((D, D))],
        out_specs=_colT_spec(),
        out_shape=jax.ShapeDtypeStruct((D, NV), _f32),
    )(acct, gt, x0t, dv, w)


def _final_body(acc_ref, gt_ref, x0t_ref, dv_ref, w_ref, wf_ref, bf_ref,
                out_ref, *, beta):
    h = _mid_layer(acc_ref, gt_ref, x0t_ref, dv_ref, w_ref, beta)
    out_ref[...] = jnp.dot(h.T, wf_ref[...],
                           preferred_element_type=_f32) + bf_ref[...]


def _final_stage(acct, gt, x0t, dv, w, wf, bf, beta):
    return pl.pallas_call(
        functools.partial(_final_body, beta=beta),
        grid=(_GRID,),
        in_specs=[_colT_spec(), _colT_spec(), _colT_spec(), _dinv_spec(),
                  _full_spec((D, D)), _full_spec((D, D)), _full_spec((1, D))],
        out_specs=_row_spec(),
        out_shape=jax.ShapeDtypeStruct((NV, D), _f32),
    )(acct, gt, x0t, dv, w, wf, bf)


@jax.jit
def _run(x, edge_index, W_init, b_init, W_gcn, W_final, b_final):
    ei = edge_index.astype(jnp.int32)
    src = ei[0]
    dst = ei[1]
    x_pad = jnp.pad(x, ((0, NV - N), (0, 0)))

    degp = _deg_kernel(dst)
    x0t, gt, dv = _init_stage(x_pad, W_init, b_init.reshape(1, D), degp)

    betas = [math.log(THETA / (i + 1) + 1.0) for i in range(L)]
    for i in range(L - 1):
        acct = _prop_kernel(gt, src, dst)
        gt = _layer_stage(acct, gt, x0t, dv, W_gcn[i], betas[i])
    acct = _prop_kernel(gt, src, dst)
    out = _final_stage(acct, gt, x0t, dv, W_gcn[L - 1], W_final,
                       b_final.reshape(1, D), betas[L - 1])
    return out[:N]


def kernel(x, edge_index, edge_weight, W_init, b_init, W_gcn, W_final,
           b_final):
    del edge_weight
    return _run(x, edge_index, W_init, b_init, W_gcn, W_final, b_final)


# speedup vs baseline: 12.1647x; 2.1312x over previous
import functools
import math

import jax
import jax.numpy as jnp
from jax import lax
from jax.experimental import pallas as pl
from jax.experimental.pallas import tpu as pltpu
from jax.experimental.pallas import tpu_sc as plsc

N = 10000
E = 320000
D = 128
L = 4
ALPHA = 0.1
THETA = 0.5

NV = 10112
NT = 32
FPT = D // NT
CB = 3200
NCB = E // CB
UNROLL = 4
EPW = E // NT
DCB = 2000
DCH = EPW // DCB

_mesh = plsc.VectorSubcoreMesh(core_axis_name="c", subcore_axis_name="s")
_f32 = jnp.float32
_sc_params = pltpu.CompilerParams(needs_layout_passes=False)


@functools.partial(
    pl.kernel,
    mesh=_mesh,
    out_type=jax.ShapeDtypeStruct((NT, NV), _f32),
    scratch_types=[
        pltpu.VMEM((NV,), _f32),
        pltpu.VMEM((DCB,), jnp.int32)
    ],
    compiler_params=_sc_params,
)
def _deg_kernel(dst_hbm, out_hbm, acc, didx):
    c = lax.axis_index("c")
    s = lax.axis_index("s")
    wid = c * 16 + s
    base = wid * EPW

    def _zero(i, carry):
        acc[pl.ds(i * 16, 16)] = jnp.zeros((16,), _f32)
        return carry

    lax.fori_loop(0, NV // 16, _zero, 0)
    ones16 = jnp.full((16,), 1.0, _f32)

    def _chunk(k, carry):
        pltpu.sync_copy(dst_hbm.at[pl.ds(base + k * DCB, DCB)], didx)

        def _edge(t, carry2):
            d16 = didx[pl.ds(t * 16, 16)]
            plsc.addupdate_scatter(acc, [d16], ones16)
            return carry2

        lax.fori_loop(0, DCB // 16, _edge, 0)
        return carry

    lax.fori_loop(0, DCH, _chunk, 0)
    pltpu.sync_copy(acc, out_hbm.at[wid])


@functools.partial(
    pl.kernel,
    mesh=_mesh,
    out_type=jax.ShapeDtypeStruct((D, NV), _f32),
    scratch_types=[
        pltpu.VMEM((FPT, NV), _f32),
        pltpu.VMEM((FPT, NV), _f32),
        pltpu.VMEM((CB,), jnp.int32),
        pltpu.VMEM((CB,), jnp.int32),
        pltpu.VMEM((CB,), jnp.int32),
        pltpu.VMEM((CB,), jnp.int32),
        pltpu.SemaphoreType.DMA,
        pltpu.SemaphoreType.DMA,
        pltpu.SemaphoreType.DMA,
        pltpu.SemaphoreType.DMA,
    ],
    compiler_params=_sc_params,
)
def _prop_kernel(g_hbm, src_hbm, dst_hbm, out_hbm,
                 gv, acc, s0, s1, d0, d1, ss0, ss1, ds0, ds1):
    c = lax.axis_index("c")
    s = lax.axis_index("s")
    wid = c * 16 + s
    sbufs = (s0, s1)
    dbufs = (d0, d1)
    ssems = (ss0, ss1)
    dsems = (ds0, ds1)

    pltpu.sync_copy(g_hbm.at[pl.ds(wid * FPT, FPT)], gv)

    def _zero(i, carry):
        for f in range(FPT):
            acc[f, pl.ds(i * 16, 16)] = jnp.zeros((16,), _f32)
        return carry

    lax.fori_loop(0, NV // 16, _zero, 0)

    f16s = [jnp.full((16,), f, jnp.int32) for f in range(FPT)]

    for b in range(2):
        pltpu.make_async_copy(
            src_hbm.at[pl.ds(b * CB, CB)], sbufs[b], ssems[b]).start()
        pltpu.make_async_copy(
            dst_hbm.at[pl.ds(b * CB, CB)], dbufs[b], dsems[b]).start()

    def _consume(b, k):
        pltpu.make_async_copy(
            src_hbm.at[pl.ds(k * CB, CB)], sbufs[b], ssems[b]).wait()
        pltpu.make_async_copy(
            dst_hbm.at[pl.ds(k * CB, CB)], dbufs[b], dsems[b]).wait()

        def _edge(t, carry2):
            off = t * (16 * UNROLL)
            svecs = [sbufs[b][pl.ds(off + u * 16, 16)] for u in range(UNROLL)]
            dvecs = [dbufs[b][pl.ds(off + u * 16, 16)] for u in range(UNROLL)]
            vals = [[plsc.load_gather(gv, [f16s[f], svecs[u]])
                     for f in range(FPT)] for u in range(UNROLL)]
            for u in range(UNROLL):
                for f in range(FPT):
                    plsc.addupdate_scatter(acc, [f16s[f], dvecs[u]],
                                           vals[u][f])
            return carry2

        lax.fori_loop(0, CB // (16 * UNROLL), _edge, 0)

    def _pair(j, carry):
        base = j * 2
        for b in range(2):
            k = base + b
            _consume(b, k)
            pltpu.make_async_copy(
                src_hbm.at[pl.ds((k + 2) * CB, CB)], sbufs[b],
                ssems[b]).start()
            pltpu.make_async_copy(
                dst_hbm.at[pl.ds((k + 2) * CB, CB)], dbufs[b],
                dsems[b]).start()
        return carry

    lax.fori_loop(0, NCB // 2 - 1, _pair, 0)
    for b in range(2):
        _consume(b, NCB - 2 + b)

    pltpu.sync_copy(acc, out_hbm.at[pl.ds(wid * FPT, FPT)])


_GRID = NV // 128


def _colT_spec():
    return pl.BlockSpec((D, 128), lambda i: (0, i))


def _row_spec():
    return pl.BlockSpec((128, D), lambda i: (i, 0))


def _dinv_spec():
    return pl.BlockSpec((1, 128), lambda i: (0, i))


def _full_spec(shape):
    nd = len(shape)
    return pl.BlockSpec(shape, lambda i: (0,) * nd)


def _init_body(x_ref, w_ref, b_ref, dg_ref, x0t_ref, gt_ref, dv_ref):
    x0 = jnp.dot(x_ref[...], w_ref[...],
                 preferred_element_type=_f32) + b_ref[...]
    x0t = x0.T
    deg = jnp.sum(dg_ref[...], axis=0) + 1.0
    dinv = (1.0 / jnp.sqrt(deg))[None, :]
    x0t_ref[...] = x0t
    dv_ref[...] = dinv
    gt_ref[...] = x0t * dinv


def _init_stage(x, w, b, degp):
    return pl.pallas_call(
        _init_body,
        grid=(_GRID,),
        in_specs=[_row_spec(), _full_spec((D, D)), _full_spec((1, D)),
                  pl.BlockSpec((NT, 128), lambda i: (0, i))],
        out_specs=[_colT_spec(), _colT_spec(), _dinv_spec()],
        out_shape=[jax.ShapeDtypeStruct((D, NV), _f32),
                   jax.ShapeDtypeStruct((D, NV), _f32),
                   jax.ShapeDtypeStruct((1, NV), _f32)],
    )(x, w, b, degp)


def _mid_layer(acc_ref, gt_ref, x0t_ref, dv_ref, w_ref, beta):
    dinv = dv_ref[...]
    p = (acc_ref[...] + gt_ref[...]) * dinv
    z = (1.0 - ALPHA) * p + ALPHA * x0t_ref[...]
    wz = lax.dot_general(w_ref[...], z, (((0,), (0,)), ((), ())),
                         preferred_element_type=_f32)
    return jnp.maximum((1.0 - beta) * z + beta * wz, 0.0)


def _layer_body(acc_ref, gt_ref, x0t_ref, dv_ref, w_ref, gout_ref, *, beta):
    h = _mid_layer(acc_ref, gt_ref, x0t_ref, dv_ref, w_ref, beta)
    gout_ref[...] = h * dv_ref[...]


def _layer_stage(acct, gt, x0t, dv, w, beta):
    return pl.pallas_call(
        functools.partial(_layer_body, beta=beta),
        grid=(_GRID,),
        in_specs=[_colT_spec(), _colT_spec(), _colT_spec(), _dinv_spec(),
                  _full_spec((D, D))],
        out_specs=_colT_spec(),
        out_shape=jax.ShapeDtypeStruct((D, NV), _f32),
    )(acct, gt, x0t, dv, w)


def _final_body(acc_ref, gt_ref, x0t_ref, dv_ref, w_ref, wf_ref, bf_ref,
                out_ref, *, beta):
    h = _mid_layer(acc_ref, gt_ref, x0t_ref, dv_ref, w_ref, beta)
    out_ref[...] = jnp.dot(h.T, wf_ref[...],
                           preferred_element_type=_f32) + bf_ref[...]


def _final_stage(acct, gt, x0t, dv, w, wf, bf, beta):
    return pl.pallas_call(
        functools.partial(_final_body, beta=beta),
        grid=(_GRID,),
        in_specs=[_colT_spec(), _colT_spec(), _colT_spec(), _dinv_spec(),
                  _full_spec((D, D)), _full_spec((D, D)), _full_spec((1, D))],
        out_specs=_row_spec(),
        out_shape=jax.ShapeDtypeStruct((NV, D), _f32),
    )(acct, gt, x0t, dv, w, wf, bf)


@jax.jit
def _run(x, edge_index, W_init, b_init, W_gcn, W_final, b_final):
    ei = edge_index.astype(jnp.int32)
    src = ei[0]
    dst = ei[1]
    x_pad = jnp.pad(x, ((0, NV - N), (0, 0)))

    degp = _deg_kernel(dst)
    x0t, gt, dv = _init_stage(x_pad, W_init, b_init.reshape(1, D), degp)

    betas = [math.log(THETA / (i + 1) + 1.0) for i in range(L)]
    for i in range(L - 1):
        acct = _prop_kernel(gt, src, dst)
        gt = _layer_stage(acct, gt, x0t, dv, W_gcn[i], betas[i])
    acct = _prop_kernel(gt, src, dst)
    out = _final_stage(acct, gt, x0t, dv, W_gcn[L - 1], W_final,
                       b_final.reshape(1, D), betas[L - 1])
    return out[:N]


def kernel(x, edge_index, edge_weight, W_init, b_init, W_gcn, W_final,
           b_final):
    del edge_weight
    return _run(x, edge_index, W_init, b_init, W_gcn, W_final, b_final)


# speedup vs baseline: 12.5980x; 1.0356x over previous
import functools
import math

import jax
import jax.numpy as jnp
from jax import lax
from jax.experimental import pallas as pl
from jax.experimental.pallas import tpu as pltpu
from jax.experimental.pallas import tpu_sc as plsc

N = 10000
E = 320000
D = 128
L = 4
ALPHA = 0.1
THETA = 0.5

NV = 10112
NT = 32
FPT = D // NT
CB = 3200
NCB = E // CB
UNROLL = 8
EPW = E // NT
DCB = 2000
DCH = EPW // DCB

_mesh = plsc.VectorSubcoreMesh(core_axis_name="c", subcore_axis_name="s")
_f32 = jnp.float32
_sc_params = pltpu.CompilerParams(needs_layout_passes=False)


@functools.partial(
    pl.kernel,
    mesh=_mesh,
    out_type=jax.ShapeDtypeStruct((NT, NV), _f32),
    scratch_types=[
        pltpu.VMEM((NV,), _f32),
        pltpu.VMEM((DCB,), jnp.int32)
    ],
    compiler_params=_sc_params,
)
def _deg_kernel(dst_hbm, out_hbm, acc, didx):
    c = lax.axis_index("c")
    s = lax.axis_index("s")
    wid = c * 16 + s
    base = wid * EPW

    def _zero(i, carry):
        acc[pl.ds(i * 16, 16)] = jnp.zeros((16,), _f32)
        return carry

    lax.fori_loop(0, NV // 16, _zero, 0)
    ones16 = jnp.full((16,), 1.0, _f32)

    def _chunk(k, carry):
        pltpu.sync_copy(dst_hbm.at[pl.ds(base + k * DCB, DCB)], didx)

        def _edge(t, carry2):
            d16 = didx[pl.ds(t * 16, 16)]
            plsc.addupdate_scatter(acc, [d16], ones16)
            return carry2

        lax.fori_loop(0, DCB // 16, _edge, 0)
        return carry

    lax.fori_loop(0, DCH, _chunk, 0)
    pltpu.sync_copy(acc, out_hbm.at[wid])


@functools.partial(
    pl.kernel,
    mesh=_mesh,
    out_type=jax.ShapeDtypeStruct((D, NV), _f32),
    scratch_types=[
        pltpu.VMEM((FPT, NV), _f32),
        pltpu.VMEM((FPT, NV), _f32),
        pltpu.VMEM((CB,), jnp.int32),
        pltpu.VMEM((CB,), jnp.int32),
        pltpu.VMEM((CB,), jnp.int32),
        pltpu.VMEM((CB,), jnp.int32),
        pltpu.SemaphoreType.DMA,
        pltpu.SemaphoreType.DMA,
        pltpu.SemaphoreType.DMA,
        pltpu.SemaphoreType.DMA,
    ],
    compiler_params=_sc_params,
)
def _prop_kernel(g_hbm, src_hbm, dst_hbm, out_hbm,
                 gv, acc, s0, s1, d0, d1, ss0, ss1, ds0, ds1):
    c = lax.axis_index("c")
    s = lax.axis_index("s")
    wid = c * 16 + s
    sbufs = (s0, s1)
    dbufs = (d0, d1)
    ssems = (ss0, ss1)
    dsems = (ds0, ds1)

    pltpu.sync_copy(g_hbm.at[pl.ds(wid * FPT, FPT)], gv)

    def _zero(i, carry):
        for f in range(FPT):
            acc[f, pl.ds(i * 16, 16)] = jnp.zeros((16,), _f32)
        return carry

    lax.fori_loop(0, NV // 16, _zero, 0)

    f16s = [jnp.full((16,), f, jnp.int32) for f in range(FPT)]

    for b in range(2):
        pltpu.make_async_copy(
            src_hbm.at[pl.ds(b * CB, CB)], sbufs[b], ssems[b]).start()
        pltpu.make_async_copy(
            dst_hbm.at[pl.ds(b * CB, CB)], dbufs[b], dsems[b]).start()

    def _consume(b, k):
        pltpu.make_async_copy(
            src_hbm.at[pl.ds(k * CB, CB)], sbufs[b], ssems[b]).wait()
        pltpu.make_async_copy(
            dst_hbm.at[pl.ds(k * CB, CB)], dbufs[b], dsems[b]).wait()

        def _edge(t, carry2):
            off = t * (16 * UNROLL)
            svecs = [sbufs[b][pl.ds(off + u * 16, 16)] for u in range(UNROLL)]
            dvecs = [dbufs[b][pl.ds(off + u * 16, 16)] for u in range(UNROLL)]
            vals = [[plsc.load_gather(gv, [f16s[f], svecs[u]])
                     for f in range(FPT)] for u in range(UNROLL)]
            for u in range(UNROLL):
                for f in range(FPT):
                    plsc.addupdate_scatter(acc, [f16s[f], dvecs[u]],
                                           vals[u][f])
            return carry2

        lax.fori_loop(0, CB // (16 * UNROLL), _edge, 0)

    def _pair(j, carry):
        base = j * 2
        for b in range(2):
            k = base + b
            _consume(b, k)
            pltpu.make_async_copy(
                src_hbm.at[pl.ds((k + 2) * CB, CB)], sbufs[b],
                ssems[b]).start()
            pltpu.make_async_copy(
                dst_hbm.at[pl.ds((k + 2) * CB, CB)], dbufs[b],
                dsems[b]).start()
        return carry

    lax.fori_loop(0, NCB // 2 - 1, _pair, 0)
    for b in range(2):
        _consume(b, NCB - 2 + b)

    pltpu.sync_copy(acc, out_hbm.at[pl.ds(wid * FPT, FPT)])


_GRID = NV // 128


def _colT_spec():
    return pl.BlockSpec((D, 128), lambda i: (0, i))


def _row_spec():
    return pl.BlockSpec((128, D), lambda i: (i, 0))


def _dinv_spec():
    return pl.BlockSpec((1, 128), lambda i: (0, i))


def _full_spec(shape):
    nd = len(shape)
    return pl.BlockSpec(shape, lambda i: (0,) * nd)


def _init_body(x_ref, w_ref, b_ref, dg_ref, x0t_ref, gt_ref, dv_ref):
    x0 = jnp.dot(x_ref[...], w_ref[...],
                 preferred_element_type=_f32) + b_ref[...]
    x0t = x0.T
    deg = jnp.sum(dg_ref[...], axis=0) + 1.0
    dinv = (1.0 / jnp.sqrt(deg))[None, :]
    x0t_ref[...] = x0t
    dv_ref[...] = dinv
    gt_ref[...] = x0t * dinv


def _init_stage(x, w, b, degp):
    return pl.pallas_call(
        _init_body,
        grid=(_GRID,),
        in_specs=[_row_spec(), _full_spec((D, D)), _full_spec((1, D)),
                  pl.BlockSpec((NT, 128), lambda i: (0, i))],
        out_specs=[_colT_spec(), _colT_spec(), _dinv_spec()],
        out_shape=[jax.ShapeDtypeStruct((D, NV), _f32),
                   jax.ShapeDtypeStruct((D, NV), _f32),
                   jax.ShapeDtypeStruct((1, NV), _f32)],
    )(x, w, b, degp)


def _mid_layer(acc_ref, gt_ref, x0t_ref, dv_ref, w_ref, beta):
    dinv = dv_ref[...]
    p = (acc_ref[...] + gt_ref[...]) * dinv
    z = (1.0 - ALPHA) * p + ALPHA * x0t_ref[...]
    wz = lax.dot_general(w_ref[...], z, (((0,), (0,)), ((), ())),
                         preferred_element_type=_f32)
    return jnp.maximum((1.0 - beta) * z + beta * wz, 0.0)


def _layer_body(acc_ref, gt_ref, x0t_ref, dv_ref, w_ref, gout_ref, *, beta):
    h = _mid_layer(acc_ref, gt_ref, x0t_ref, dv_ref, w_ref, beta)
    gout_ref[...] = h * dv_ref[...]


def _layer_stage(acct, gt, x0t, dv, w, beta):
    return pl.pallas_call(
        functools.partial(_layer_body, beta=beta),
        grid=(_GRID,),
        in_specs=[_colT_spec(), _colT_spec(), _colT_spec(), _dinv_spec(),
                  _full_spec((D, D))],
        out_specs=_colT_spec(),
        out_shape=jax.ShapeDtypeStruct((D, NV), _f32),
    )(acct, gt, x0t, dv, w)


def _final_body(acc_ref, gt_ref, x0t_ref, dv_ref, w_ref, wf_ref, bf_ref,
                out_ref, *, beta):
    h = _mid_layer(acc_ref, gt_ref, x0t_ref, dv_ref, w_ref, beta)
    out_ref[...] = jnp.dot(h.T, wf_ref[...],
                           preferred_element_type=_f32) + bf_ref[...]


def _final_stage(acct, gt, x0t, dv, w, wf, bf, beta):
    return pl.pallas_call(
        functools.partial(_final_body, beta=beta),
        grid=(_GRID,),
        in_specs=[_colT_spec(), _colT_spec(), _colT_spec(), _dinv_spec(),
                  _full_spec((D, D)), _full_spec((D, D)), _full_spec((1, D))],
        out_specs=_row_spec(),
        out_shape=jax.ShapeDtypeStruct((NV, D), _f32),
    )(acct, gt, x0t, dv, w, wf, bf)


@jax.jit
def _run(x, edge_index, W_init, b_init, W_gcn, W_final, b_final):
    ei = edge_index.astype(jnp.int32)
    src = ei[0]
    dst = ei[1]
    x_pad = jnp.pad(x, ((0, NV - N), (0, 0)))

    degp = _deg_kernel(dst)
    x0t, gt, dv = _init_stage(x_pad, W_init, b_init.reshape(1, D), degp)

    betas = [math.log(THETA / (i + 1) + 1.0) for i in range(L)]
    for i in range(L - 1):
        acct = _prop_kernel(gt, src, dst)
        gt = _layer_stage(acct, gt, x0t, dv, W_gcn[i], betas[i])
    acct = _prop_kernel(gt, src, dst)
    out = _final_stage(acct, gt, x0t, dv, W_gcn[L - 1], W_final,
                       b_final.reshape(1, D), betas[L - 1])
    return out[:N]


def kernel(x, edge_index, edge_weight, W_init, b_init, W_gcn, W_final,
           b_final):
    del edge_weight
    return _run(x, edge_index, W_init, b_init, W_gcn, W_final, b_final)


# speedup vs baseline: 12.6258x; 1.0022x over previous
import functools
import math

import jax
import jax.numpy as jnp
from jax import lax
from jax.experimental import pallas as pl
from jax.experimental.pallas import tpu as pltpu
from jax.experimental.pallas import tpu_sc as plsc

N = 10000
E = 320000
D = 128
L = 4
ALPHA = 0.1
THETA = 0.5

NV = 10112
NT = 32
FPT = D // NT
CB = 3200
NCB = E // CB
UNROLL = 10
EPW = E // NT
DCB = 2000
DCH = EPW // DCB

_mesh = plsc.VectorSubcoreMesh(core_axis_name="c", subcore_axis_name="s")
_f32 = jnp.float32
_sc_params = pltpu.CompilerParams(needs_layout_passes=False)


@functools.partial(
    pl.kernel,
    mesh=_mesh,
    out_type=jax.ShapeDtypeStruct((NT, NV), _f32),
    scratch_types=[
        pltpu.VMEM((NV,), _f32),
        pltpu.VMEM((DCB,), jnp.int32)
    ],
    compiler_params=_sc_params,
)
def _deg_kernel(dst_hbm, out_hbm, acc, didx):
    c = lax.axis_index("c")
    s = lax.axis_index("s")
    wid = c * 16 + s
    base = wid * EPW

    def _zero(i, carry):
        acc[pl.ds(i * 16, 16)] = jnp.zeros((16,), _f32)
        return carry

    lax.fori_loop(0, NV // 16, _zero, 0)
    ones16 = jnp.full((16,), 1.0, _f32)

    def _chunk(k, carry):
        pltpu.sync_copy(dst_hbm.at[pl.ds(base + k * DCB, DCB)], didx)

        def _edge(t, carry2):
            d16 = didx[pl.ds(t * 16, 16)]
            plsc.addupdate_scatter(acc, [d16], ones16)
            return carry2

        lax.fori_loop(0, DCB // 16, _edge, 0)
        return carry

    lax.fori_loop(0, DCH, _chunk, 0)
    pltpu.sync_copy(acc, out_hbm.at[wid])


@functools.partial(
    pl.kernel,
    mesh=_mesh,
    out_type=jax.ShapeDtypeStruct((D, NV), _f32),
    scratch_types=[
        pltpu.VMEM((FPT, NV), _f32),
        pltpu.VMEM((FPT, NV), _f32),
        pltpu.VMEM((CB,), jnp.int32),
        pltpu.VMEM((CB,), jnp.int32),
        pltpu.VMEM((CB,), jnp.int32),
        pltpu.VMEM((CB,), jnp.int32),
        pltpu.SemaphoreType.DMA,
        pltpu.SemaphoreType.DMA,
        pltpu.SemaphoreType.DMA,
        pltpu.SemaphoreType.DMA,
    ],
    compiler_params=_sc_params,
)
def _prop_kernel(g_hbm, src_hbm, dst_hbm, out_hbm,
                 gv, acc, s0, s1, d0, d1, ss0, ss1, ds0, ds1):
    c = lax.axis_index("c")
    s = lax.axis_index("s")
    wid = c * 16 + s
    sbufs = (s0, s1)
    dbufs = (d0, d1)
    ssems = (ss0, ss1)
    dsems = (ds0, ds1)

    pltpu.sync_copy(g_hbm.at[pl.ds(wid * FPT, FPT)], gv)

    def _zero(i, carry):
        for f in range(FPT):
            acc[f, pl.ds(i * 16, 16)] = jnp.zeros((16,), _f32)
        return carry

    lax.fori_loop(0, NV // 16, _zero, 0)

    f16s = [jnp.full((16,), f, jnp.int32) for f in range(FPT)]

    for b in range(2):
        pltpu.make_async_copy(
            src_hbm.at[pl.ds(b * CB, CB)], sbufs[b], ssems[b]).start()
        pltpu.make_async_copy(
            dst_hbm.at[pl.ds(b * CB, CB)], dbufs[b], dsems[b]).start()

    def _consume(b, k):
        pltpu.make_async_copy(
            src_hbm.at[pl.ds(k * CB, CB)], sbufs[b], ssems[b]).wait()
        pltpu.make_async_copy(
            dst_hbm.at[pl.ds(k * CB, CB)], dbufs[b], dsems[b]).wait()

        def _edge(t, carry2):
            off = t * (16 * UNROLL)
            svecs = [sbufs[b][pl.ds(off + u * 16, 16)] for u in range(UNROLL)]
            dvecs = [dbufs[b][pl.ds(off + u * 16, 16)] for u in range(UNROLL)]
            vals = [[plsc.load_gather(gv, [f16s[f], svecs[u]])
                     for f in range(FPT)] for u in range(UNROLL)]
            for u in range(UNROLL):
                for f in range(FPT):
                    plsc.addupdate_scatter(acc, [f16s[f], dvecs[u]],
                                           vals[u][f])
            return carry2

        lax.fori_loop(0, CB // (16 * UNROLL), _edge, 0)

    def _pair(j, carry):
        base = j * 2
        for b in range(2):
            k = base + b
            _consume(b, k)
            pltpu.make_async_copy(
                src_hbm.at[pl.ds((k + 2) * CB, CB)], sbufs[b],
                ssems[b]).start()
            pltpu.make_async_copy(
                dst_hbm.at[pl.ds((k + 2) * CB, CB)], dbufs[b],
                dsems[b]).start()
        return carry

    lax.fori_loop(0, NCB // 2 - 1, _pair, 0)
    for b in range(2):
        _consume(b, NCB - 2 + b)

    pltpu.sync_copy(acc, out_hbm.at[pl.ds(wid * FPT, FPT)])


_GRID = NV // 128


def _colT_spec():
    return pl.BlockSpec((D, 128), lambda i: (0, i))


def _row_spec():
    return pl.BlockSpec((128, D), lambda i: (i, 0))


def _dinv_spec():
    return pl.BlockSpec((1, 128), lambda i: (0, i))


def _full_spec(shape):
    nd = len(shape)
    return pl.BlockSpec(shape, lambda i: (0,) * nd)


def _init_body(x_ref, w_ref, b_ref, dg_ref, x0t_ref, gt_ref, dv_ref):
    x0 = jnp.dot(x_ref[...], w_ref[...],
                 preferred_element_type=_f32) + b_ref[...]
    x0t = x0.T
    deg = jnp.sum(dg_ref[...], axis=0) + 1.0
    dinv = (1.0 / jnp.sqrt(deg))[None, :]
    x0t_ref[...] = x0t
    dv_ref[...] = dinv
    gt_ref[...] = x0t * dinv


def _init_stage(x, w, b, degp):
    return pl.pallas_call(
        _init_body,
        grid=(_GRID,),
        in_specs=[_row_spec(), _full_spec((D, D)), _full_spec((1, D)),
                  pl.BlockSpec((NT, 128), lambda i: (0, i))],
        out_specs=[_colT_spec(), _colT_spec(), _dinv_spec()],
        out_shape=[jax.ShapeDtypeStruct((D, NV), _f32),
                   jax.ShapeDtypeStruct((D, NV), _f32),
                   jax.ShapeDtypeStruct((1, NV), _f32)],
    )(x, w, b, degp)


def _mid_layer(acc_ref, gt_ref, x0t_ref, dv_ref, w_ref, beta):
    dinv = dv_ref[...]
    p = (acc_ref[...] + gt_ref[...]) * dinv
    z = (1.0 - ALPHA) * p + ALPHA * x0t_ref[...]
    wz = lax.dot_general(w_ref[...], z, (((0,), (0,)), ((), ())),
                         preferred_element_type=_f32)
    return jnp.maximum((1.0 - beta) * z + beta * wz, 0.0)


def _layer_body(acc_ref, gt_ref, x0t_ref, dv_ref, w_ref, gout_ref, *, beta):
    h = _mid_layer(acc_ref, gt_ref, x0t_ref, dv_ref, w_ref, beta)
    gout_ref[...] = h * dv_ref[...]


def _layer_stage(acct, gt, x0t, dv, w, beta):
    return pl.pallas_call(
        functools.partial(_layer_body, beta=beta),
        grid=(_GRID,),
        in_specs=[_colT_spec(), _colT_spec(), _colT_spec(), _dinv_spec(),
                  _full_spec((D, D))],
        out_specs=_colT_spec(),
        out_shape=jax.ShapeDtypeStruct((D, NV), _f32),
    )(acct, gt, x0t, dv, w)


def _final_body(acc_ref, gt_ref, x0t_ref, dv_ref, w_ref, wf_ref, bf_ref,
                out_ref, *, beta):
    h = _mid_layer(acc_ref, gt_ref, x0t_ref, dv_ref, w_ref, beta)
    out_ref[...] = jnp.dot(h.T, wf_ref[...],
                           preferred_element_type=_f32) + bf_ref[...]


def _final_stage(acct, gt, x0t, dv, w, wf, bf, beta):
    return pl.pallas_call(
        functools.partial(_final_body, beta=beta),
        grid=(_GRID,),
        in_specs=[_colT_spec(), _colT_spec(), _colT_spec(), _dinv_spec(),
                  _full_spec((D, D)), _full_spec((D, D)), _full_spec((1, D))],
        out_specs=_row_spec(),
        out_shape=jax.ShapeDtypeStruct((NV, D), _f32),
    )(acct, gt, x0t, dv, w, wf, bf)


@jax.jit
def _run(x, edge_index, W_init, b_init, W_gcn, W_final, b_final):
    ei = edge_index.astype(jnp.int32)
    src = ei[0]
    dst = ei[1]
    x_pad = jnp.pad(x, ((0, NV - N), (0, 0)))

    degp = _deg_kernel(dst)
    x0t, gt, dv = _init_stage(x_pad, W_init, b_init.reshape(1, D), degp)

    betas = [math.log(THETA / (i + 1) + 1.0) for i in range(L)]
    for i in range(L - 1):
        acct = _prop_kernel(gt, src, dst)
        gt = _layer_stage(acct, gt, x0t, dv, W_gcn[i], betas[i])
    acct = _prop_kernel(gt, src, dst)
    out = _final_stage(acct, gt, x0t, dv, W_gcn[L - 1], W_final,
                       b_final.reshape(1, D), betas[L - 1])
    return out[:N]


def kernel(x, edge_index, edge_weight, W_init, b_init, W_gcn, W_final,
           b_final):
    del edge_weight
    return _run(x, edge_index, W_init, b_init, W_gcn, W_final, b_final)
